# Initial kernel scaffold; baseline (speedup 1.0000x reference)
import jax
import jax.numpy as jnp
from jax.experimental import pallas as pl


def kernel(x, y):
    raise NotImplementedError("write your pallas kernel here")


# speedup vs baseline: 11.2222x; 11.2222x over previous
import functools

import jax
import jax.numpy as jnp
from jax.experimental import pallas as pl

_K = 16
_BLOCK_N = 256


def _knn_body(a_ref, bt_ref, x2_ref, y2_ref, out_ref):
    a = a_ref[0]
    bt = bt_ref[0]
    x2 = x2_ref[0]
    y2 = y2_ref[0]
    inner = jax.lax.dot_general(
        a, bt, (((1,), (0,)), ((), ())),
        preferred_element_type=jnp.float32)
    dist = (x2 + (-2.0) * inner) + y2
    bn, m = dist.shape
    col = jax.lax.broadcasted_iota(jnp.int32, (bn, m), 1)
    kcol = jax.lax.broadcasted_iota(jnp.int32, (bn, _K), 1)
    out = jnp.zeros((bn, _K), jnp.int32)
    inf = jnp.float32(jnp.inf)
    for k in range(_K):
        mval = jnp.min(dist, axis=1, keepdims=True)
        idx = jnp.min(jnp.where(dist == mval, col, m), axis=1)
        out = jnp.where(kcol == k, idx[:, None], out)
        dist = jnp.where(col == idx[:, None], inf, dist)
    out_ref[0] = out


def _normalize(v, axis):
    n = jnp.sqrt(jnp.sum(v * v, axis=axis, keepdims=True))
    return v / jnp.maximum(n, 1e-12)


@jax.jit
def kernel(x, y):
    xn = _normalize(x, 1)[..., 0]
    yn = _normalize(y, 1)[..., 0]
    xt = jnp.transpose(xn, (0, 2, 1))
    b, n, c = xt.shape
    m = yn.shape[2]
    x2 = jnp.sum(xt * xt, axis=-1, keepdims=True)
    y2 = jnp.sum(yn * yn, axis=1, keepdims=True)

    grid = (b, n // _BLOCK_N)
    nn_idx = pl.pallas_call(
        _knn_body,
        grid=grid,
        in_specs=[
            pl.BlockSpec((1, _BLOCK_N, c), lambda i, j: (i, j, 0)),
            pl.BlockSpec((1, c, m), lambda i, j: (i, 0, 0)),
            pl.BlockSpec((1, _BLOCK_N, 1), lambda i, j: (i, j, 0)),
            pl.BlockSpec((1, 1, m), lambda i, j: (i, 0, 0)),
        ],
        out_specs=pl.BlockSpec((1, _BLOCK_N, _K), lambda i, j: (i, j, 0)),
        out_shape=jax.ShapeDtypeStruct((b, n, _K), jnp.int32),
    )(xt, yn, x2, y2)

    center_idx = jnp.broadcast_to(
        jnp.arange(n, dtype=nn_idx.dtype)[None, :, None], (b, n, _K))
    return jnp.stack((nn_idx, center_idx), axis=0)


# speedup vs baseline: 17.6832x; 1.5757x over previous
import functools

import jax
import jax.numpy as jnp
from jax.experimental import pallas as pl

_K = 16
_BLOCK_N = 256
_R = 6
_LANES = 128


def _knn_body(a_ref, bt_ref, x2_ref, y2_ref, out_ref):
    a = a_ref[0]
    bt = bt_ref[0]
    x2 = x2_ref[0]
    y2 = y2_ref[0]
    inner = jax.lax.dot_general(
        a, bt, (((1,), (0,)), ((), ())),
        preferred_element_type=jnp.float32)
    dist = (x2 + (-2.0) * inner) + y2
    bn, m = dist.shape
    ngroups = m // _LANES
    inf = jnp.float32(jnp.inf)
    lane = jax.lax.broadcasted_iota(jnp.int32, (bn, _LANES), 1)

    vals = [jnp.full((bn, _LANES), inf, jnp.float32) for _ in range(_R)]
    cols = [jnp.full((bn, _LANES), m, jnp.int32) for _ in range(_R)]
    for g in range(ngroups):
        t = dist[:, g * _LANES:(g + 1) * _LANES]
        tc = lane + (g * _LANES)
        for j in range(_R):
            c = t < vals[j]
            vals[j], t = jnp.where(c, t, vals[j]), jnp.where(c, vals[j], t)
            cols[j], tc = jnp.where(c, tc, cols[j]), jnp.where(c, cols[j], tc)

    kcol = jax.lax.broadcasted_iota(jnp.int32, (bn, _K), 1)
    out = jnp.zeros((bn, _K), jnp.int32)
    for k in range(_K):
        gv = jnp.min(vals[0], axis=1, keepdims=True)
        eq = vals[0] == gv
        win = jnp.min(jnp.where(eq, cols[0], m), axis=1)
        out = jnp.where(kcol == k, win[:, None], out)
        pop = eq & (cols[0] == win[:, None])
        for j in range(_R - 1):
            vals[j] = jnp.where(pop, vals[j + 1], vals[j])
            cols[j] = jnp.where(pop, cols[j + 1], cols[j])
        vals[_R - 1] = jnp.where(pop, inf, vals[_R - 1])
        cols[_R - 1] = jnp.where(pop, m, cols[_R - 1])
    out_ref[0] = out


def _normalize(v, axis):
    n = jnp.sqrt(jnp.sum(v * v, axis=axis, keepdims=True))
    return v / jnp.maximum(n, 1e-12)


@jax.jit
def kernel(x, y):
    xn = _normalize(x, 1)[..., 0]
    yn = _normalize(y, 1)[..., 0]
    xt = jnp.transpose(xn, (0, 2, 1))
    b, n, c = xt.shape
    m = yn.shape[2]
    x2 = jnp.sum(xt * xt, axis=-1, keepdims=True)
    y2 = jnp.sum(yn * yn, axis=1, keepdims=True)

    grid = (b, n // _BLOCK_N)
    nn_idx = pl.pallas_call(
        _knn_body,
        grid=grid,
        in_specs=[
            pl.BlockSpec((1, _BLOCK_N, c), lambda i, j: (i, j, 0)),
            pl.BlockSpec((1, c, m), lambda i, j: (i, 0, 0)),
            pl.BlockSpec((1, _BLOCK_N, 1), lambda i, j: (i, j, 0)),
            pl.BlockSpec((1, 1, m), lambda i, j: (i, 0, 0)),
        ],
        out_specs=pl.BlockSpec((1, _BLOCK_N, _K), lambda i, j: (i, j, 0)),
        out_shape=jax.ShapeDtypeStruct((b, n, _K), jnp.int32),
    )(xt, yn, x2, y2)

    center_idx = jnp.broadcast_to(
        jnp.arange(n, dtype=nn_idx.dtype)[None, :, None], (b, n, _K))
    return jnp.stack((nn_idx, center_idx), axis=0)
